# baseline (device time: 12693 ns/iter reference)
import jax
import jax.numpy as jnp
from jax import lax
from jax.experimental import pallas as pl
from jax.experimental.pallas import tpu as pltpu

C = 4
SCALE_ROWS = 8


def kernel(x):
    m, n = x.shape
    rows = m // C

    def body(x_ref, out_ref, qsend, qrecv, ssend, srecv, sems):
        my_x = lax.axis_index("x")
        my_y = lax.axis_index("y")
        my_z = lax.axis_index("z")
        z_peer = (my_x, my_y, 1 - my_z)

        barrier_sem = pltpu.get_barrier_semaphore()
        pl.semaphore_signal(
            barrier_sem, inc=1, device_id=z_peer,
            device_id_type=pl.DeviceIdType.MESH,
        )
        pl.semaphore_wait(barrier_sem, 1)

        for c in range(C):
            s = jnp.max(jnp.abs(x_ref[pl.ds(c * rows, rows), :])) / 127.0
            ssend[c, :] = jnp.full((n,), jnp.maximum(s, 1e-30), jnp.float32)

        scale_rdma = pltpu.make_async_remote_copy(
            src_ref=ssend, dst_ref=srecv,
            send_sem=sems.at[0], recv_sem=sems.at[1],
            device_id=z_peer, device_id_type=pl.DeviceIdType.MESH,
        )
        scale_rdma.start()

        rdmas = []
        for c in range(C):
            sl = pl.ds(c * rows, rows)
            inv = 127.0 / jnp.maximum(ssend[c, 0] * 127.0, 1e-30)
            qsend[sl, :] = jnp.clip(
                jnp.round(x_ref[sl, :] * inv), -127.0, 127.0
            ).astype(jnp.int8)
            r = pltpu.make_async_remote_copy(
                src_ref=qsend.at[sl],
                dst_ref=qrecv.at[sl],
                send_sem=sems.at[2 + 2 * c],
                recv_sem=sems.at[3 + 2 * c],
                device_id=z_peer,
                device_id_type=pl.DeviceIdType.MESH,
            )
            r.start()
            rdmas.append(r)

        scale_rdma.wait_recv()
        for c in range(C):
            sl = pl.ds(c * rows, rows)
            rdmas[c].wait_recv()
            peer_scale = srecv[c, 0]
            out_ref[sl, :] = (
                x_ref[sl, :] + qrecv[sl, :].astype(jnp.float32) * peer_scale
            ).astype(jnp.bfloat16)

        scale_rdma.wait_send()
        for c in range(C):
            rdmas[c].wait_send()

    return pl.pallas_call(
        body,
        out_shape=jax.ShapeDtypeStruct((m, n), jnp.bfloat16),
        in_specs=[pl.BlockSpec(memory_space=pltpu.VMEM)],
        out_specs=pl.BlockSpec(memory_space=pltpu.VMEM),
        scratch_shapes=[
            pltpu.VMEM((m, n), jnp.int8),
            pltpu.VMEM((m, n), jnp.int8),
            pltpu.VMEM((SCALE_ROWS, n), jnp.float32),
            pltpu.VMEM((SCALE_ROWS, n), jnp.float32),
            pltpu.SemaphoreType.DMA((2 + 2 * C,)),
        ],
        compiler_params=pltpu.CompilerParams(collective_id=0),
    )(x)
